# baseline (speedup 1.0000x reference)
import functools

import jax
import jax.numpy as jnp
from jax import lax
from jax.experimental import pallas as pl
from jax.experimental.pallas import tpu as pltpu
from jax.experimental.pallas import tpu_sc as plsc

N = 10000
E = 320000
D = 128
ED = 16
HG = 2048
NE = 8
K = 4


_NC = 2
_NS = 16
_C = 128
_SLOTS = 160
_NB = 4
_ACCN = N + 16
_RPS = 624
_RCH = 104
_REM = _ACCN - _RPS * _NS
_WREM = N - _RPS * _NS
_DH = D // _NC
_W = _DH + ED


def _sc_segment_sums(xz0, xz1, eidx, ea_p, zeros_d):
    mesh = plsc.VectorSubcoreMesh(core_axis_name="c", subcore_axis_name="s")

    @functools.partial(
        pl.kernel,
        mesh=mesh,
        out_type=jax.ShapeDtypeStruct((_NC, N, _W), jnp.float32),
        scratch_types=[
        ] + [pltpu.VMEM((2, _C), jnp.int32) for _ in range(_NB)] + [
            pltpu.VMEM((_C, _W), jnp.float32) for _ in range(_NB)] + [
            pltpu.VMEM((_C, ED), jnp.float32) for _ in range(_NB)] + [
            pltpu.VMEM((_RCH, _W), jnp.float32),
            pltpu.VMEM_SHARED((_ACCN, _W), jnp.float32),
            pltpu.SemaphoreType.DMA((_NB,)),
            pltpu.SemaphoreType.DMA((_NB,)),
            pltpu.SemaphoreType.DMA((_NB,)),
            pltpu.SemaphoreType.DMA((_NB,)),
        ],
        compiler_params=pltpu.CompilerParams(use_tc_tiling_on_sc=False),
    )
    def k(xz0_hbm, xz1_hbm, ei_hbm, ea_hbm, zd_hbm,
          axp_hbm,
          i0, i1, i2, i3, g0, g1, g2_, g3, e0, e1, e2, e3,
          stg_d, accx, si, sg, se, sx):
        idx = (i0, i1, i2, i3)
        xr = (g0, g1, g2_, g3)
        eab = (e0, e1, e2, e3)
        c = lax.axis_index("c")
        s = lax.axis_index("s")

        for j in range(_RPS // _RCH):
            ro = s * _RPS + j * _RCH
            pltpu.sync_copy(zd_hbm.at[pl.ds(ro, _RCH)], stg_d)
            pltpu.sync_copy(stg_d, accx.at[pl.ds(ro, _RCH)])

        @pl.when(s == 0)
        def _():
            pltpu.sync_copy(stg_d.at[pl.ds(0, _REM)],
                            accx.at[pl.ds(_RPS * _NS, _REM)])

        plsc.subcore_barrier()

        xh_hbm = (xz0_hbm, xz1_hbm)

        def run_core(cc):
            xh = xh_hbm[cc]

            def cid(t):
                return s + t * _NS

            def ebase(t):
                return jnp.minimum(cid(t), E // _C - 1) * _C

            def start_idx(b, t):
                pltpu.async_copy(ei_hbm.at[:, pl.ds(ebase(t), _C)],
                                 idx[b], si.at[b])

            def wait_idx(b):
                pltpu.make_async_copy(ei_hbm.at[:, pl.ds(0, _C)],
                                      idx[b], si.at[b]).wait()

            def start_gather(b, t):
                pltpu.async_copy(xh.at[idx[b].at[0]], xr[b], sg.at[b])
                pltpu.async_copy(ea_hbm.at[pl.ds(ebase(t), _C)], eab[b],
                                 se.at[b])

            def wait_gather(b):
                pltpu.make_async_copy(xh.at[idx[b].at[0]], xr[b],
                                      sg.at[b]).wait()
                pltpu.make_async_copy(ea_hbm.at[pl.ds(0, _C)], eab[b],
                                      se.at[b]).wait()

            def start_scatter(b):
                pltpu.async_copy(xr[b], accx.at[idx[b].at[1]], sx.at[b],
                                 add=True)

            def wait_scatter(b):
                pltpu.make_async_copy(xr[b], accx.at[idx[b].at[1]],
                                      sx.at[b]).wait()

            def merge_ea(b):
                xrb = xr[b]
                eb = eab[b]

                def body(r, carry):
                    xrb[r, pl.ds(_DH, ED)] = eb[r, :]
                    return carry

                lax.fori_loop(0, _C, body, 0)

            def slot(t, r, drain=True, g2=True, i3=True):
                b = r % _NB
                q2 = (r + 2) % _NB
                q3 = (r + 3) % _NB
                wait_gather(b)
                merge_ea(b)
                start_scatter(b)
                if drain:
                    wait_scatter(q3)
                if g2:
                    wait_idx(q2)

                    @pl.when(cid(t + 2) >= E // _C)
                    def _():
                        for jj in range(_C // 16):
                            idx[q2][1, pl.ds(jj * 16, 16)] = N + lax.iota(
                                jnp.int32, 16)

                    start_gather(q2, t + 2)
                if i3:
                    start_idx(q3, t + 3)

            for t in range(3):
                start_idx(t, t)
            for t in range(2):
                wait_idx(t)
                start_gather(t, t)
            for t in range(_NB):
                slot(t, t, drain=(t >= 1))

            def body(i, carry):
                t0 = _NB * i
                for r in range(_NB):
                    slot(t0 + r, r)
                return carry

            lax.fori_loop(1, _SLOTS // _NB - 1, body, 0)

            for t in range(_SLOTS - _NB, _SLOTS):
                slot(t, t % _NB, g2=(t + 2 < _SLOTS), i3=(t + 3 < _SLOTS))
            wait_scatter((_SLOTS - 1) % _NB)

        @pl.when(c == 0)
        def _():
            run_core(0)

        @pl.when(c == 1)
        def _():
            run_core(1)

        plsc.subcore_barrier()

        for j in range(_RPS // _RCH):
            ro = s * _RPS + j * _RCH
            pltpu.sync_copy(accx.at[pl.ds(ro, _RCH)], stg_d)
            pltpu.sync_copy(stg_d, axp_hbm.at[c, pl.ds(ro, _RCH)])

        @pl.when(s == 0)
        def _():
            pltpu.sync_copy(accx.at[pl.ds(_RPS * _NS, _WREM)],
                            stg_d.at[pl.ds(0, _WREM)])
            pltpu.sync_copy(stg_d.at[pl.ds(0, _WREM)],
                            axp_hbm.at[c, pl.ds(_RPS * _NS, _WREM)])

    return k(xz0, xz1, eidx, ea_p, zeros_d)


_BN = 2000


def _dense_kernel(x_ref, axp_ref, ws_ref, wm_ref, we_ref, bh_ref,
                  wog_ref, bog_ref, out_ref):
    xb = x_ref[...]
    ax = jnp.concatenate([axp_ref[0, :, :_DH], axp_ref[1, :, :_DH]],
                         axis=-1)
    ea = axp_ref[0, :, _DH:]
    acc = jnp.broadcast_to(bog_ref[...], xb.shape)
    for k in range(K):
        pre = (jnp.dot(xb, ws_ref[k], preferred_element_type=jnp.float32)
               + jnp.dot(ax, wm_ref[k], preferred_element_type=jnp.float32)
               + jnp.dot(ea, we_ref[k], preferred_element_type=jnp.float32)
               + bh_ref[k])
        h = jnp.maximum(pre, 0.0)
        acc = acc + jnp.dot(h, wog_ref[k], preferred_element_type=jnp.float32)
    out_ref[...] = acc


def _dense(x, axp, ws, wm, we, bh, wog, bog):
    grid = (N // _BN,)
    return pl.pallas_call(
        _dense_kernel,
        grid=grid,
        in_specs=[
            pl.BlockSpec((_BN, D), lambda i: (i, 0)),
            pl.BlockSpec((_NC, _BN, _W), lambda i: (0, i, 0)),
            pl.BlockSpec((K, D, D), lambda i: (0, 0, 0)),
            pl.BlockSpec((K, D, D), lambda i: (0, 0, 0)),
            pl.BlockSpec((K, ED, D), lambda i: (0, 0, 0)),
            pl.BlockSpec((K, 1, D), lambda i: (0, 0, 0)),
            pl.BlockSpec((K, D, D), lambda i: (0, 0, 0)),
            pl.BlockSpec((1, D), lambda i: (0, 0)),
        ],
        out_specs=pl.BlockSpec((_BN, D), lambda i: (i, 0)),
        out_shape=jax.ShapeDtypeStruct((N, D), jnp.float32),
        compiler_params=pltpu.CompilerParams(
            dimension_semantics=("parallel",)),
    )(x, axp, ws, wm, we, bh, wog, bog)


def kernel(x, edge_index, edge_attr, W_body, b_body, W_gate, b_gate,
           W_noise, b_noise, W_self, W_msg, W_edge, b_h, W_out, b_out):
    zeros_d = jnp.zeros((N, _W), jnp.float32)
    zc = jnp.zeros((N, ED), jnp.float32)
    xz0 = jnp.concatenate([x[:, :_DH], zc], axis=1)
    xz1 = jnp.concatenate([x[:, _DH:], zc], axis=1)
    axp = _sc_segment_sums(xz0, xz1, edge_index, edge_attr, zeros_d)

    g = jnp.mean(x, axis=0)
    hg = jax.nn.relu(g @ W_body + b_body)
    logits = hg @ W_gate + b_gate

    top_v, top_i = jax.lax.top_k(logits, K)
    gates = jax.nn.softmax(top_v)

    ws = W_self[top_i]
    wm = W_msg[top_i]
    we = W_edge[top_i]
    bh = b_h[top_i].reshape(K, 1, D)
    wog = W_out[top_i] * gates[:, None, None]
    bog = (gates[:, None] * b_out[top_i]).sum(axis=0).reshape(1, D)

    return _dense(x, axp, ws, wm, we, bh, wog, bog)
